# baseline (speedup 1.0000x reference)
import functools

import jax
import jax.numpy as jnp
from jax import lax
from jax.experimental import pallas as pl
from jax.experimental.pallas import tpu as pltpu
from jax.experimental.pallas import tpu_sc as plsc

NUM_ENT = 1000000
NUM_REL = 1000
EMB_DIM = 32
BATCH = 16384

NC = 2
NS = 16
NW = NC * NS
BPW = BATCH // NW
CH = 32
NCH = BPW // CH

_mesh = plsc.VectorSubcoreMesh(core_axis_name="c", subcore_axis_name="s")


@functools.partial(
    pl.kernel,
    mesh=_mesh,
    compiler_params=pltpu.CompilerParams(
        needs_layout_passes=False, use_tc_tiling_on_sc=False),
    out_type=jax.ShapeDtypeStruct((BATCH,), jnp.float32),
    scratch_types=[
        pltpu.VMEM((NCH, CH), jnp.int32),
        pltpu.VMEM((NCH, CH), jnp.int32),
        pltpu.VMEM((NCH, CH), jnp.int32),
        pltpu.VMEM((EMB_DIM, 8 * CH), jnp.float32),
        pltpu.VMEM((EMB_DIM, 8 * CH), jnp.float32),
        pltpu.VMEM((EMB_DIM, 8 * CH), jnp.float32),
        pltpu.VMEM((EMB_DIM, 8 * CH), jnp.float32),
        pltpu.VMEM((EMB_DIM, NUM_REL), jnp.float32),
        pltpu.VMEM((EMB_DIM, NUM_REL), jnp.float32),
        pltpu.VMEM((BPW,), jnp.float32),
        pltpu.SemaphoreType.DMA,
        pltpu.SemaphoreType.DMA,
    ],
)
def _simple_score(heads_h, rels_h, tails_h, ent_h, ent_t, rel, rel_inv,
                  out_h, hidx, ridx, tidx, hh, ht, th, tt, rv, riv,
                  outv, sem, rsem):
    wid = lax.axis_index("s") * NC + lax.axis_index("c")

    rel_cp = pltpu.async_copy(rel, rv, rsem)
    rel_inv_cp = pltpu.async_copy(rel_inv, riv, rsem)
    pltpu.sync_copy(heads_h.at[wid], hidx)
    pltpu.sync_copy(rels_h.at[wid], ridx)
    pltpu.sync_copy(tails_h.at[wid], tidx)
    rel_cp.wait()
    rel_inv_cp.wait()

    lanes = lax.iota(jnp.int32, 16)
    iota8 = lanes * 8

    def chunk(c, carry):
        for g in range(CH // 16):
            vh = hidx[c, pl.ds(g * 16, 16)]
            vt = tidx[c, pl.ds(g * 16, 16)]
            for j in range(16):
                s = g * 16 + j
                e = pl.multiple_of((vh[j] >> 3) * 8, 8)
                dcol = pl.ds(8 * s, 8)
                pltpu.async_copy(ent_h.at[:, pl.ds(e, 8)], hh.at[:, dcol], sem)
                pltpu.async_copy(ent_t.at[:, pl.ds(e, 8)], th.at[:, dcol], sem)
                e2 = pl.multiple_of((vt[j] >> 3) * 8, 8)
                pltpu.async_copy(ent_h.at[:, pl.ds(e2, 8)], ht.at[:, dcol], sem)
                pltpu.async_copy(ent_t.at[:, pl.ds(e2, 8)], tt.at[:, dcol], sem)

        def drain(i, carry2):
            pltpu.make_async_copy(ent_h.at[:, pl.ds(0, 8)],
                                  hh.at[:, pl.ds(0, 8)], sem).wait()
            return carry2

        lax.fori_loop(0, CH * 4, drain, 0)

        for g in range(CH // 16):
            vh = hidx[c, pl.ds(g * 16, 16)]
            vt = tidx[c, pl.ds(g * 16, 16)]
            vr = ridx[c, pl.ds(g * 16, 16)]
            colh = g * 128 + iota8 + (vh & 7)
            colt = g * 128 + iota8 + (vt & 7)
            facc = jnp.zeros((16,), jnp.float32)
            iacc = jnp.zeros((16,), jnp.float32)
            for d in range(EMB_DIM):
                dvec = jnp.full((16,), d, jnp.int32)
                fh = plsc.load_gather(hh, [dvec, colh])
                ft = plsc.load_gather(tt, [dvec, colt])
                fr = plsc.load_gather(rv, [dvec, vr])
                facc = facc + fh * fr * ft
                ih = plsc.load_gather(ht, [dvec, colt])
                it = plsc.load_gather(th, [dvec, colh])
                ir = plsc.load_gather(riv, [dvec, vr])
                iacc = iacc + ih * ir * it
            score = (facc + iacc) * 0.5
            score = jnp.minimum(jnp.maximum(score, -20.0), 20.0)
            outv[pl.ds(c * CH + g * 16, 16)] = score
        return carry

    lax.fori_loop(0, NCH, chunk, 0)

    pltpu.sync_copy(outv, out_h.at[pl.ds(wid * BPW, BPW)])


def kernel(heads, rels, tails, ent_h_embs, ent_t_embs, rel_embs, rel_inv_embs):
    shape3 = (NW, NCH, CH)
    return _simple_score(heads.reshape(shape3), rels.reshape(shape3),
                         tails.reshape(shape3),
                         ent_h_embs.T, ent_t_embs.T,
                         rel_embs.T, rel_inv_embs.T)


# speedup vs baseline: 2.6981x; 2.6981x over previous
import functools

import jax
import jax.numpy as jnp
from jax import lax
from jax.experimental import pallas as pl
from jax.experimental.pallas import tpu as pltpu
from jax.experimental.pallas import tpu_sc as plsc

NUM_ENT = 1000000
NUM_REL = 1000
EMB_DIM = 32
BATCH = 16384
PACK = 128 // EMB_DIM
NPACKED = NUM_ENT // PACK

NC = 2
NS = 16
NW = NC * NS
BPW = BATCH // NW
CHUNK = 128
NCHUNK = BPW // CHUNK
GPC = CHUNK // 16

NTCOL = (NUM_ENT + 127) // 128

_mesh = plsc.VectorSubcoreMesh(core_axis_name="c", subcore_axis_name="s")


@functools.partial(
    pl.kernel,
    mesh=_mesh,
    compiler_params=pltpu.CompilerParams(needs_layout_passes=False),
    out_type=(jax.ShapeDtypeStruct((NPACKED, 128), jnp.float32),
              jax.ShapeDtypeStruct((NPACKED, 128), jnp.float32)),
    scratch_types=[
        pltpu.VMEM((EMB_DIM, 128), jnp.float32),
        pltpu.VMEM((EMB_DIM, 128), jnp.float32),
        pltpu.VMEM((32, 128), jnp.float32),
        pltpu.VMEM((32, 128), jnp.float32),
        pltpu.SemaphoreType.DMA,
    ],
)
def _pack(ent_h, ent_t, tail_h, tail_t, out_h, out_t,
          slab_h, slab_t, pk_h, pk_t, sem):
    wid = lax.axis_index("s") * NC + lax.axis_index("c")
    ncols = jnp.where(wid < 4, (NTCOL - 1) // 32 + 1, (NTCOL - 1) // 32)
    lanes = lax.iota(jnp.int32, 16)

    def col(i, carry):
        c = wid + i * 32
        off = pl.multiple_of(c * 128, 128)
        cp_h = pltpu.async_copy(ent_h.at[:, pl.ds(off, 128)], slab_h, sem)
        cp_t = pltpu.async_copy(ent_t.at[:, pl.ds(off, 128)], slab_t, sem)
        cp_h.wait()
        cp_t.wait()
        for r in range(32):
            for m in range(8):
                dvec = lanes + 16 * (m & 1)
                cvec = jnp.full((16,), 4 * r + (m >> 1), jnp.int32)
                pk_h[r, pl.ds(16 * m, 16)] = plsc.load_gather(
                    slab_h, [dvec, cvec])
                pk_t[r, pl.ds(16 * m, 16)] = plsc.load_gather(
                    slab_t, [dvec, cvec])
        base = pl.multiple_of(c * 32, 8)
        pltpu.sync_copy(pk_h, out_h.at[pl.ds(base, 32)])
        pltpu.sync_copy(pk_t, out_t.at[pl.ds(base, 32)])
        return carry

    lax.fori_loop(0, ncols, col, 0)

    @pl.when(wid == 4)
    def _():
        pltpu.sync_copy(tail_h, pk_h.at[pl.ds(0, 16)])
        pltpu.sync_copy(tail_t, pk_t.at[pl.ds(0, 16)])
        pltpu.sync_copy(pk_h.at[pl.ds(0, 16)],
                        out_h.at[pl.ds(NPACKED - 16, 16)])
        pltpu.sync_copy(pk_t.at[pl.ds(0, 16)],
                        out_t.at[pl.ds(NPACKED - 16, 16)])


@functools.partial(
    pl.kernel,
    mesh=_mesh,
    compiler_params=pltpu.CompilerParams(needs_layout_passes=False),
    out_type=jax.ShapeDtypeStruct((BATCH,), jnp.float32),
    scratch_types=[
        pltpu.VMEM((NCHUNK, CHUNK), jnp.int32),
        pltpu.VMEM((NCHUNK, CHUNK), jnp.int32),
        pltpu.VMEM((NCHUNK, CHUNK), jnp.int32),
        pltpu.VMEM((NCHUNK, CHUNK), jnp.int32),
        pltpu.VMEM((NCHUNK, CHUNK), jnp.int32),
        pltpu.VMEM((NCHUNK, CHUNK), jnp.int32),
        pltpu.VMEM((CHUNK, 128), jnp.float32),
        pltpu.VMEM((CHUNK, 128), jnp.float32),
        pltpu.VMEM((CHUNK, 128), jnp.float32),
        pltpu.VMEM((CHUNK, 128), jnp.float32),
        pltpu.VMEM((CHUNK, 128), jnp.float32),
        pltpu.VMEM((CHUNK, 128), jnp.float32),
        pltpu.VMEM((BPW,), jnp.float32),
        pltpu.SemaphoreType.DMA,
    ],
)
def _score(hdiv_h, hcol_h, rdiv_h, rcol_h, tdiv_h, tcol_h,
           ent_h, ent_t, rel, rel_inv,
           out_h, hdiv, hcol, rdiv, rcol, tdiv, tcol,
           hh, ht, th, tt, rv, riv, outv, sem):
    wid = lax.axis_index("s") * NC + lax.axis_index("c")

    pltpu.sync_copy(hdiv_h.at[wid], hdiv)
    pltpu.sync_copy(hcol_h.at[wid], hcol)
    pltpu.sync_copy(rdiv_h.at[wid], rdiv)
    pltpu.sync_copy(rcol_h.at[wid], rcol)
    pltpu.sync_copy(tdiv_h.at[wid], tdiv)
    pltpu.sync_copy(tcol_h.at[wid], tcol)

    lanes = lax.iota(jnp.int32, 16)

    for c in range(NCHUNK):
        copies = [
            pltpu.async_copy(ent_h.at[hdiv.at[c]], hh, sem),
            pltpu.async_copy(ent_h.at[tdiv.at[c]], ht, sem),
            pltpu.async_copy(ent_t.at[hdiv.at[c]], th, sem),
            pltpu.async_copy(ent_t.at[tdiv.at[c]], tt, sem),
            pltpu.async_copy(rel.at[rdiv.at[c]], rv, sem),
            pltpu.async_copy(rel_inv.at[rdiv.at[c]], riv, sem),
        ]
        for cp in copies:
            cp.wait()

        def group(g, carry, c=c):
            rows = g * 16 + lanes
            ch = hcol[c, pl.ds(g * 16, 16)]
            cr = rcol[c, pl.ds(g * 16, 16)]
            ct = tcol[c, pl.ds(g * 16, 16)]
            facc = jnp.zeros((16,), jnp.float32)
            iacc = jnp.zeros((16,), jnp.float32)
            for d in range(EMB_DIM):
                chd = ch + d
                crd = cr + d
                ctd = ct + d
                fh = plsc.load_gather(hh, [rows, chd])
                fr = plsc.load_gather(rv, [rows, crd])
                ft = plsc.load_gather(tt, [rows, ctd])
                facc = facc + fh * fr * ft
                ih = plsc.load_gather(ht, [rows, ctd])
                ir = plsc.load_gather(riv, [rows, crd])
                it = plsc.load_gather(th, [rows, chd])
                iacc = iacc + ih * ir * it
            score = (facc + iacc) * 0.5
            score = jnp.minimum(jnp.maximum(score, -20.0), 20.0)
            outv[pl.ds(c * CHUNK + g * 16, 16)] = score
            return carry

        lax.fori_loop(0, GPC, group, 0)

    pltpu.sync_copy(outv, out_h.at[pl.ds(wid * BPW, BPW)])


def kernel(heads, rels, tails, ent_h_embs, ent_t_embs, rel_embs, rel_inv_embs):
    tail_h = ent_h_embs[NUM_ENT - 16 * PACK:].reshape(16, 128)
    tail_t = ent_t_embs[NUM_ENT - 16 * PACK:].reshape(16, 128)
    ent_h_p, ent_t_p = _pack(ent_h_embs.T, ent_t_embs.T, tail_h, tail_t)
    shape3 = (NW, NCHUNK, CHUNK)
    hdiv = (heads // PACK).reshape(shape3)
    hcol = ((heads % PACK) * EMB_DIM).reshape(shape3)
    rdiv = (rels // PACK).reshape(shape3)
    rcol = ((rels % PACK) * EMB_DIM).reshape(shape3)
    tdiv = (tails // PACK).reshape(shape3)
    tcol = ((tails % PACK) * EMB_DIM).reshape(shape3)
    rel_p = rel_embs.reshape(NUM_REL // PACK, 128)
    rel_inv_p = rel_inv_embs.reshape(NUM_REL // PACK, 128)
    return _score(hdiv, hcol, rdiv, rcol, tdiv, tcol,
                  ent_h_p, ent_t_p, rel_p, rel_inv_p)


# speedup vs baseline: 3.1828x; 1.1797x over previous
import functools

import jax
import jax.numpy as jnp
from jax import lax
from jax.experimental import pallas as pl
from jax.experimental.pallas import tpu as pltpu
from jax.experimental.pallas import tpu_sc as plsc

NUM_ENT = 1000000
NUM_REL = 1000
EMB_DIM = 32
BATCH = 16384
PACK = 128 // EMB_DIM
NPACKED = NUM_ENT // PACK

NC = 2
NS = 16
NW = NC * NS
BPW = BATCH // NW
CHUNK = 128
NCHUNK = BPW // CHUNK
GPC = CHUNK // 16

NTCOL = (NUM_ENT + 127) // 128

_mesh = plsc.VectorSubcoreMesh(core_axis_name="c", subcore_axis_name="s")


@functools.partial(
    pl.kernel,
    mesh=_mesh,
    compiler_params=pltpu.CompilerParams(needs_layout_passes=False),
    out_type=(jax.ShapeDtypeStruct((NPACKED, 128), jnp.float32),
              jax.ShapeDtypeStruct((NPACKED, 128), jnp.float32)),
    scratch_types=[
        pltpu.VMEM((EMB_DIM, 128), jnp.float32),
        pltpu.VMEM((EMB_DIM, 128), jnp.float32),
        pltpu.VMEM((EMB_DIM, 128), jnp.float32),
        pltpu.VMEM((EMB_DIM, 128), jnp.float32),
        pltpu.VMEM((32, 128), jnp.float32),
        pltpu.VMEM((32, 128), jnp.float32),
        pltpu.VMEM((32, 128), jnp.float32),
        pltpu.VMEM((32, 128), jnp.float32),
        pltpu.SemaphoreType.DMA,
        pltpu.SemaphoreType.DMA,
    ],
)
def _pack(ent_h, ent_t, tail_h, tail_t, out_h, out_t,
          sa_h, sa_t, sb_h, sb_t, pa_h, pa_t, pb_h, pb_t, rd, wr):
    wid = lax.axis_index("s") * NC + lax.axis_index("c")
    lanes = lax.iota(jnp.int32, 16)
    nfull = NTCOL - 1

    def read(c, sl_h, sl_t):
        off = pl.multiple_of(c * 128, 128)
        pltpu.async_copy(ent_h.at[:, pl.ds(off, 128)], sl_h, rd)
        pltpu.async_copy(ent_t.at[:, pl.ds(off, 128)], sl_t, rd)

    def wait_reads():
        pltpu.make_async_copy(ent_h.at[:, pl.ds(0, 128)], sa_h, rd).wait()
        pltpu.make_async_copy(ent_h.at[:, pl.ds(0, 128)], sa_t, rd).wait()

    def wait_writes():
        for _ in range(4):
            pltpu.make_async_copy(pa_h, out_h.at[pl.ds(0, 32)], wr).wait()

    def transpose(sl_h, sl_t, dst_h, dst_t):
        for r in range(32):
            for m in range(8):
                dvec = lanes + 16 * (m & 1)
                cvec = jnp.full((16,), 4 * r + (m >> 1), jnp.int32)
                dst_h[r, pl.ds(16 * m, 16)] = plsc.load_gather(
                    sl_h, [dvec, cvec])
                dst_t[r, pl.ds(16 * m, 16)] = plsc.load_gather(
                    sl_t, [dvec, cvec])

    def write(c, src_h, src_t):
        base = pl.multiple_of(c * 32, 8)
        pltpu.async_copy(src_h, out_h.at[pl.ds(base, 32)], wr)
        pltpu.async_copy(src_t, out_t.at[pl.ds(base, 32)], wr)

    read(wid, sa_h, sa_t)
    write(wid, pa_h, pa_t)
    write(wid, pb_h, pb_t)

    def step(i, carry):
        ca = wid + (2 * i) * 32
        cb = wid + (2 * i + 1) * 32
        cn = jnp.minimum(wid + (2 * i + 2) * 32, nfull - 1)
        read(cb, sb_h, sb_t)
        wait_reads()
        wait_writes()
        transpose(sa_h, sa_t, pa_h, pa_t)
        write(ca, pa_h, pa_t)
        read(cn, sa_h, sa_t)
        wait_reads()
        transpose(sb_h, sb_t, pb_h, pb_t)
        write(cb, pb_h, pb_t)
        return carry

    lax.fori_loop(0, 122, step, 0)
    wait_reads()
    wait_writes()

    @pl.when(wid < 4)
    def _():
        c = wid + 244 * 32
        read(c, sa_h, sa_t)
        wait_reads()
        transpose(sa_h, sa_t, pa_h, pa_t)
        write(c, pa_h, pa_t)
        pltpu.make_async_copy(pa_h, out_h.at[pl.ds(0, 32)], wr).wait()
        pltpu.make_async_copy(pa_h, out_h.at[pl.ds(0, 32)], wr).wait()

    @pl.when(wid == 4)
    def _():
        cp_h = pltpu.async_copy(tail_h, pa_h.at[pl.ds(0, 16)], rd)
        cp_t = pltpu.async_copy(tail_t, pa_t.at[pl.ds(0, 16)], rd)
        cp_h.wait()
        cp_t.wait()
        pltpu.sync_copy(pa_h.at[pl.ds(0, 16)],
                        out_h.at[pl.ds(NPACKED - 16, 16)])
        pltpu.sync_copy(pa_t.at[pl.ds(0, 16)],
                        out_t.at[pl.ds(NPACKED - 16, 16)])


@functools.partial(
    pl.kernel,
    mesh=_mesh,
    compiler_params=pltpu.CompilerParams(needs_layout_passes=False),
    out_type=jax.ShapeDtypeStruct((BATCH,), jnp.float32),
    scratch_types=[
        pltpu.VMEM((NCHUNK, CHUNK), jnp.int32),
        pltpu.VMEM((NCHUNK, CHUNK), jnp.int32),
        pltpu.VMEM((NCHUNK, CHUNK), jnp.int32),
        pltpu.VMEM((NCHUNK, CHUNK), jnp.int32),
        pltpu.VMEM((NCHUNK, CHUNK), jnp.int32),
        pltpu.VMEM((NCHUNK, CHUNK), jnp.int32),
        pltpu.VMEM((CHUNK, 128), jnp.float32),
        pltpu.VMEM((CHUNK, 128), jnp.float32),
        pltpu.VMEM((CHUNK, 128), jnp.float32),
        pltpu.VMEM((CHUNK, 128), jnp.float32),
        pltpu.VMEM((CHUNK, 128), jnp.float32),
        pltpu.VMEM((CHUNK, 128), jnp.float32),
        pltpu.VMEM((BPW,), jnp.float32),
        pltpu.SemaphoreType.DMA,
    ],
)
def _score(hdiv_h, hcol_h, rdiv_h, rcol_h, tdiv_h, tcol_h,
           ent_h, ent_t, rel, rel_inv,
           out_h, hdiv, hcol, rdiv, rcol, tdiv, tcol,
           hh, ht, th, tt, rv, riv, outv, sem):
    wid = lax.axis_index("s") * NC + lax.axis_index("c")

    pltpu.sync_copy(hdiv_h.at[wid], hdiv)
    pltpu.sync_copy(hcol_h.at[wid], hcol)
    pltpu.sync_copy(rdiv_h.at[wid], rdiv)
    pltpu.sync_copy(rcol_h.at[wid], rcol)
    pltpu.sync_copy(tdiv_h.at[wid], tdiv)
    pltpu.sync_copy(tcol_h.at[wid], tcol)

    lanes = lax.iota(jnp.int32, 16)

    for c in range(NCHUNK):
        copies = [
            pltpu.async_copy(ent_h.at[hdiv.at[c]], hh, sem),
            pltpu.async_copy(ent_h.at[tdiv.at[c]], ht, sem),
            pltpu.async_copy(ent_t.at[hdiv.at[c]], th, sem),
            pltpu.async_copy(ent_t.at[tdiv.at[c]], tt, sem),
            pltpu.async_copy(rel.at[rdiv.at[c]], rv, sem),
            pltpu.async_copy(rel_inv.at[rdiv.at[c]], riv, sem),
        ]
        for cp in copies:
            cp.wait()

        def group(g, carry, c=c):
            rows = g * 16 + lanes
            ch = hcol[c, pl.ds(g * 16, 16)]
            cr = rcol[c, pl.ds(g * 16, 16)]
            ct = tcol[c, pl.ds(g * 16, 16)]
            facc = jnp.zeros((16,), jnp.float32)
            iacc = jnp.zeros((16,), jnp.float32)
            for d in range(EMB_DIM):
                chd = ch + d
                crd = cr + d
                ctd = ct + d
                fh = plsc.load_gather(hh, [rows, chd])
                fr = plsc.load_gather(rv, [rows, crd])
                ft = plsc.load_gather(tt, [rows, ctd])
                facc = facc + fh * fr * ft
                ih = plsc.load_gather(ht, [rows, ctd])
                ir = plsc.load_gather(riv, [rows, crd])
                it = plsc.load_gather(th, [rows, chd])
                iacc = iacc + ih * ir * it
            score = (facc + iacc) * 0.5
            score = jnp.minimum(jnp.maximum(score, -20.0), 20.0)
            outv[pl.ds(c * CHUNK + g * 16, 16)] = score
            return carry

        lax.fori_loop(0, GPC, group, 0)

    pltpu.sync_copy(outv, out_h.at[pl.ds(wid * BPW, BPW)])


def kernel(heads, rels, tails, ent_h_embs, ent_t_embs, rel_embs, rel_inv_embs):
    tail_h = ent_h_embs[NUM_ENT - 16 * PACK:].reshape(16, 128)
    tail_t = ent_t_embs[NUM_ENT - 16 * PACK:].reshape(16, 128)
    ent_h_p, ent_t_p = _pack(ent_h_embs.T, ent_t_embs.T, tail_h, tail_t)
    shape3 = (NW, NCHUNK, CHUNK)
    hdiv = (heads // PACK).reshape(shape3)
    hcol = ((heads % PACK) * EMB_DIM).reshape(shape3)
    rdiv = (rels // PACK).reshape(shape3)
    rcol = ((rels % PACK) * EMB_DIM).reshape(shape3)
    tdiv = (tails // PACK).reshape(shape3)
    tcol = ((tails % PACK) * EMB_DIM).reshape(shape3)
    rel_p = rel_embs.reshape(NUM_REL // PACK, 128)
    rel_inv_p = rel_inv_embs.reshape(NUM_REL // PACK, 128)
    return _score(hdiv, hcol, rdiv, rcol, tdiv, tcol,
                  ent_h_p, ent_t_p, rel_p, rel_inv_p)


# speedup vs baseline: 6.2003x; 1.9480x over previous
import functools

import jax
import jax.numpy as jnp
from jax import lax
from jax.experimental import pallas as pl
from jax.experimental.pallas import tpu as pltpu
from jax.experimental.pallas import tpu_sc as plsc

NUM_ENT = 1000000
NUM_REL = 1000
EMB_DIM = 32
BATCH = 16384
PACK = 128 // EMB_DIM
NPACKED = NUM_ENT // PACK

NC = 2
NS = 16
NW = NC * NS
BPW = BATCH // NW
CHUNK = 128
NCHUNK = BPW // CHUNK
GPC = CHUNK // 16

NTCOL = (NUM_ENT + 127) // 128

_mesh = plsc.VectorSubcoreMesh(core_axis_name="c", subcore_axis_name="s")


@functools.partial(
    pl.kernel,
    mesh=_mesh,
    compiler_params=pltpu.CompilerParams(needs_layout_passes=False),
    out_type=(jax.ShapeDtypeStruct((NPACKED, 128), jnp.float32),
              jax.ShapeDtypeStruct((NPACKED, 128), jnp.float32)),
    scratch_types=[
        pltpu.VMEM((EMB_DIM, 128), jnp.float32),
        pltpu.VMEM((EMB_DIM, 128), jnp.float32),
        pltpu.VMEM((EMB_DIM, 128), jnp.float32),
        pltpu.VMEM((EMB_DIM, 128), jnp.float32),
        pltpu.VMEM((32, 128), jnp.float32),
        pltpu.VMEM((32, 128), jnp.float32),
        pltpu.VMEM((32, 128), jnp.float32),
        pltpu.VMEM((32, 128), jnp.float32),
        pltpu.SemaphoreType.DMA,
        pltpu.SemaphoreType.DMA,
    ],
)
def _pack(ent_h, ent_t, tail_h, tail_t, out_h, out_t,
          sa_h, sa_t, sb_h, sb_t, pa_h, pa_t, pb_h, pb_t, rd, wr):
    wid = lax.axis_index("s") * NC + lax.axis_index("c")
    lanes = lax.iota(jnp.int32, 16)
    nfull = NTCOL - 1

    def read(c, sl_h, sl_t):
        off = pl.multiple_of(c * 128, 128)
        pltpu.async_copy(ent_h.at[:, pl.ds(off, 128)], sl_h, rd)
        pltpu.async_copy(ent_t.at[:, pl.ds(off, 128)], sl_t, rd)

    def wait_reads():
        pltpu.make_async_copy(ent_h.at[:, pl.ds(0, 128)], sa_h, rd).wait()
        pltpu.make_async_copy(ent_h.at[:, pl.ds(0, 128)], sa_t, rd).wait()

    def wait_writes():
        for _ in range(4):
            pltpu.make_async_copy(pa_h, out_h.at[pl.ds(0, 32)], wr).wait()

    dveq = lanes >> 2
    jveq = lanes & 3

    def transpose(sl_h, sl_t, dst_h, dst_t):
        for r in range(32):
            for m in range(8):
                dvec = dveq + 4 * m
                cvec = jveq + 4 * r
                dst_h[r, pl.ds(16 * m, 16)] = plsc.load_gather(
                    sl_h, [dvec, cvec])
                dst_t[r, pl.ds(16 * m, 16)] = plsc.load_gather(
                    sl_t, [dvec, cvec])

    def write(c, src_h, src_t):
        base = pl.multiple_of(c * 32, 8)
        pltpu.async_copy(src_h, out_h.at[pl.ds(base, 32)], wr)
        pltpu.async_copy(src_t, out_t.at[pl.ds(base, 32)], wr)

    read(wid, sa_h, sa_t)
    write(wid, pa_h, pa_t)
    write(wid, pb_h, pb_t)

    def step(i, carry):
        ca = wid + (2 * i) * 32
        cb = wid + (2 * i + 1) * 32
        cn = jnp.minimum(wid + (2 * i + 2) * 32, nfull - 1)
        read(cb, sb_h, sb_t)
        wait_reads()
        wait_writes()
        transpose(sa_h, sa_t, pa_h, pa_t)
        write(ca, pa_h, pa_t)
        read(cn, sa_h, sa_t)
        wait_reads()
        transpose(sb_h, sb_t, pb_h, pb_t)
        write(cb, pb_h, pb_t)
        return carry

    lax.fori_loop(0, 122, step, 0)
    wait_reads()
    wait_writes()

    @pl.when(wid < 4)
    def _():
        c = wid + 244 * 32
        read(c, sa_h, sa_t)
        wait_reads()
        transpose(sa_h, sa_t, pa_h, pa_t)
        write(c, pa_h, pa_t)
        pltpu.make_async_copy(pa_h, out_h.at[pl.ds(0, 32)], wr).wait()
        pltpu.make_async_copy(pa_h, out_h.at[pl.ds(0, 32)], wr).wait()

    @pl.when(wid == 4)
    def _():
        cp_h = pltpu.async_copy(tail_h, pa_h.at[pl.ds(0, 16)], rd)
        cp_t = pltpu.async_copy(tail_t, pa_t.at[pl.ds(0, 16)], rd)
        cp_h.wait()
        cp_t.wait()
        pltpu.sync_copy(pa_h.at[pl.ds(0, 16)],
                        out_h.at[pl.ds(NPACKED - 16, 16)])
        pltpu.sync_copy(pa_t.at[pl.ds(0, 16)],
                        out_t.at[pl.ds(NPACKED - 16, 16)])


@functools.partial(
    pl.kernel,
    mesh=_mesh,
    compiler_params=pltpu.CompilerParams(needs_layout_passes=False),
    out_type=jax.ShapeDtypeStruct((BATCH,), jnp.float32),
    scratch_types=[
        pltpu.VMEM((NCHUNK, CHUNK), jnp.int32),
        pltpu.VMEM((NCHUNK, CHUNK), jnp.int32),
        pltpu.VMEM((NCHUNK, CHUNK), jnp.int32),
        pltpu.VMEM((NCHUNK, CHUNK), jnp.int32),
        pltpu.VMEM((NCHUNK, CHUNK), jnp.int32),
        pltpu.VMEM((NCHUNK, CHUNK), jnp.int32),
        pltpu.VMEM((CHUNK, 128), jnp.float32),
        pltpu.VMEM((CHUNK, 128), jnp.float32),
        pltpu.VMEM((CHUNK, 128), jnp.float32),
        pltpu.VMEM((CHUNK, 128), jnp.float32),
        pltpu.VMEM((CHUNK, 128), jnp.float32),
        pltpu.VMEM((CHUNK, 128), jnp.float32),
        pltpu.VMEM((BPW,), jnp.float32),
        pltpu.SemaphoreType.DMA,
    ],
)
def _score(hdiv_h, hcol_h, rdiv_h, rcol_h, tdiv_h, tcol_h,
           ent_h, ent_t, rel, rel_inv,
           out_h, hdiv, hcol, rdiv, rcol, tdiv, tcol,
           hh, ht, th, tt, rv, riv, outv, sem):
    wid = lax.axis_index("s") * NC + lax.axis_index("c")

    pltpu.sync_copy(hdiv_h.at[wid], hdiv)
    pltpu.sync_copy(hcol_h.at[wid], hcol)
    pltpu.sync_copy(rdiv_h.at[wid], rdiv)
    pltpu.sync_copy(rcol_h.at[wid], rcol)
    pltpu.sync_copy(tdiv_h.at[wid], tdiv)
    pltpu.sync_copy(tcol_h.at[wid], tcol)

    lanes = lax.iota(jnp.int32, 16)

    for c in range(NCHUNK):
        copies = [
            pltpu.async_copy(ent_h.at[hdiv.at[c]], hh, sem),
            pltpu.async_copy(ent_h.at[tdiv.at[c]], ht, sem),
            pltpu.async_copy(ent_t.at[hdiv.at[c]], th, sem),
            pltpu.async_copy(ent_t.at[tdiv.at[c]], tt, sem),
            pltpu.async_copy(rel.at[rdiv.at[c]], rv, sem),
            pltpu.async_copy(rel_inv.at[rdiv.at[c]], riv, sem),
        ]
        for cp in copies:
            cp.wait()

        def group(g, carry, c=c):
            rows = g * 16 + lanes
            ch = hcol[c, pl.ds(g * 16, 16)]
            cr = rcol[c, pl.ds(g * 16, 16)]
            ct = tcol[c, pl.ds(g * 16, 16)]
            facc = jnp.zeros((16,), jnp.float32)
            iacc = jnp.zeros((16,), jnp.float32)
            for d in range(EMB_DIM):
                chd = ch + 4 * d
                crd = cr + 4 * d
                ctd = ct + 4 * d
                fh = plsc.load_gather(hh, [rows, chd])
                fr = plsc.load_gather(rv, [rows, crd])
                ft = plsc.load_gather(tt, [rows, ctd])
                facc = facc + fh * fr * ft
                ih = plsc.load_gather(ht, [rows, ctd])
                ir = plsc.load_gather(riv, [rows, crd])
                it = plsc.load_gather(th, [rows, chd])
                iacc = iacc + ih * ir * it
            score = (facc + iacc) * 0.5
            score = jnp.minimum(jnp.maximum(score, -20.0), 20.0)
            outv[pl.ds(c * CHUNK + g * 16, 16)] = score
            return carry

        lax.fori_loop(0, GPC, group, 0)

    pltpu.sync_copy(outv, out_h.at[pl.ds(wid * BPW, BPW)])


def kernel(heads, rels, tails, ent_h_embs, ent_t_embs, rel_embs, rel_inv_embs):
    tail_h = (ent_h_embs[NUM_ENT - 16 * PACK:].reshape(16, PACK, EMB_DIM)
              .transpose(0, 2, 1).reshape(16, 128))
    tail_t = (ent_t_embs[NUM_ENT - 16 * PACK:].reshape(16, PACK, EMB_DIM)
              .transpose(0, 2, 1).reshape(16, 128))
    ent_h_p, ent_t_p = _pack(ent_h_embs.T, ent_t_embs.T, tail_h, tail_t)
    shape3 = (NW, NCHUNK, CHUNK)
    hdiv = (heads // PACK).reshape(shape3)
    hcol = (heads % PACK).reshape(shape3)
    rdiv = (rels // PACK).reshape(shape3)
    rcol = (rels % PACK).reshape(shape3)
    tdiv = (tails // PACK).reshape(shape3)
    tcol = (tails % PACK).reshape(shape3)
    rel_p = (rel_embs.reshape(NUM_REL // PACK, PACK, EMB_DIM)
             .transpose(0, 2, 1).reshape(NUM_REL // PACK, 128))
    rel_inv_p = (rel_inv_embs.reshape(NUM_REL // PACK, PACK, EMB_DIM)
                 .transpose(0, 2, 1).reshape(NUM_REL // PACK, 128))
    return _score(hdiv, hcol, rdiv, rcol, tdiv, tcol,
                  ent_h_p, ent_t_p, rel_p, rel_inv_p)


# speedup vs baseline: 6.9969x; 1.1285x over previous
import functools

import jax
import jax.numpy as jnp
from jax import lax
from jax.experimental import pallas as pl
from jax.experimental.pallas import tpu as pltpu
from jax.experimental.pallas import tpu_sc as plsc

NUM_ENT = 1000000
NUM_REL = 1000
EMB_DIM = 32
BATCH = 16384
PACK = 128 // EMB_DIM
NPACKED = NUM_ENT // PACK

NC = 2
NS = 16
NW = NC * NS
BPW = BATCH // NW
CHUNK = 128
NCHUNK = BPW // CHUNK
GPC = CHUNK // 16

NTCOL = (NUM_ENT + 127) // 128

_mesh = plsc.VectorSubcoreMesh(core_axis_name="c", subcore_axis_name="s")


@functools.partial(
    pl.kernel,
    mesh=_mesh,
    compiler_params=pltpu.CompilerParams(needs_layout_passes=False),
    out_type=(jax.ShapeDtypeStruct((NPACKED, 128), jnp.float32),
              jax.ShapeDtypeStruct((NPACKED, 128), jnp.float32)),
    scratch_types=[
        pltpu.VMEM((EMB_DIM, 128), jnp.float32),
        pltpu.VMEM((EMB_DIM, 128), jnp.float32),
        pltpu.VMEM((EMB_DIM, 128), jnp.float32),
        pltpu.VMEM((EMB_DIM, 128), jnp.float32),
        pltpu.VMEM((32, 128), jnp.float32),
        pltpu.VMEM((32, 128), jnp.float32),
        pltpu.VMEM((32, 128), jnp.float32),
        pltpu.VMEM((32, 128), jnp.float32),
        pltpu.SemaphoreType.DMA,
        pltpu.SemaphoreType.DMA,
    ],
)
def _pack(ent_h, ent_t, tail_h, tail_t, out_h, out_t,
          sa_h, sa_t, sb_h, sb_t, pa_h, pa_t, pb_h, pb_t, rd, wr):
    wid = lax.axis_index("s") * NC + lax.axis_index("c")
    lanes = lax.iota(jnp.int32, 16)
    nfull = NTCOL - 1

    def read(c, sl_h, sl_t):
        off = pl.multiple_of(c * 128, 128)
        pltpu.async_copy(ent_h.at[:, pl.ds(off, 128)], sl_h, rd)
        pltpu.async_copy(ent_t.at[:, pl.ds(off, 128)], sl_t, rd)

    def wait_reads():
        pltpu.make_async_copy(ent_h.at[:, pl.ds(0, 128)], sa_h, rd).wait()
        pltpu.make_async_copy(ent_h.at[:, pl.ds(0, 128)], sa_t, rd).wait()

    def wait_writes():
        for _ in range(4):
            pltpu.make_async_copy(pa_h, out_h.at[pl.ds(0, 32)], wr).wait()

    dveq = lanes >> 2
    jveq = lanes & 3

    def transpose(sl_h, sl_t, dst_h, dst_t):
        for r in range(32):
            for m in range(4, 8):
                dvec = dveq + 4 * m
                cvec = jveq + 4 * r
                dst_h[r, pl.ds(16 * m, 16)] = plsc.load_gather(
                    sl_h, [dvec, cvec])
                dst_t[r, pl.ds(16 * m, 16)] = plsc.load_gather(
                    sl_t, [dvec, cvec])
        for d in range(16):
            cvec = jveq + 4 * d
            for t in range(8):
                rvec = dveq + 4 * t
                cols = pl.ds(16 * t, 16)
                plsc.store_scatter(dst_h, [rvec, cvec], sl_h[d, cols])
                plsc.store_scatter(dst_t, [rvec, cvec], sl_t[d, cols])

    def write(c, src_h, src_t):
        base = pl.multiple_of(c * 32, 8)
        pltpu.async_copy(src_h, out_h.at[pl.ds(base, 32)], wr)
        pltpu.async_copy(src_t, out_t.at[pl.ds(base, 32)], wr)

    read(wid, sa_h, sa_t)
    write(wid, pa_h, pa_t)
    write(wid, pb_h, pb_t)

    def step(i, carry):
        ca = wid + (2 * i) * 32
        cb = wid + (2 * i + 1) * 32
        cn = jnp.minimum(wid + (2 * i + 2) * 32, nfull - 1)
        read(cb, sb_h, sb_t)
        wait_reads()
        wait_writes()
        transpose(sa_h, sa_t, pa_h, pa_t)
        write(ca, pa_h, pa_t)
        read(cn, sa_h, sa_t)
        wait_reads()
        transpose(sb_h, sb_t, pb_h, pb_t)
        write(cb, pb_h, pb_t)
        return carry

    lax.fori_loop(0, 122, step, 0)
    wait_reads()
    wait_writes()

    @pl.when(wid < 4)
    def _():
        c = wid + 244 * 32
        read(c, sa_h, sa_t)
        wait_reads()
        transpose(sa_h, sa_t, pa_h, pa_t)
        write(c, pa_h, pa_t)
        pltpu.make_async_copy(pa_h, out_h.at[pl.ds(0, 32)], wr).wait()
        pltpu.make_async_copy(pa_h, out_h.at[pl.ds(0, 32)], wr).wait()

    @pl.when(wid == 4)
    def _():
        cp_h = pltpu.async_copy(tail_h, pa_h.at[pl.ds(0, 16)], rd)
        cp_t = pltpu.async_copy(tail_t, pa_t.at[pl.ds(0, 16)], rd)
        cp_h.wait()
        cp_t.wait()
        pltpu.sync_copy(pa_h.at[pl.ds(0, 16)],
                        out_h.at[pl.ds(NPACKED - 16, 16)])
        pltpu.sync_copy(pa_t.at[pl.ds(0, 16)],
                        out_t.at[pl.ds(NPACKED - 16, 16)])


@functools.partial(
    pl.kernel,
    mesh=_mesh,
    compiler_params=pltpu.CompilerParams(needs_layout_passes=False),
    out_type=jax.ShapeDtypeStruct((BATCH,), jnp.float32),
    scratch_types=[
        pltpu.VMEM((NCHUNK, CHUNK), jnp.int32),
        pltpu.VMEM((NCHUNK, CHUNK), jnp.int32),
        pltpu.VMEM((NCHUNK, CHUNK), jnp.int32),
        pltpu.VMEM((NCHUNK, CHUNK), jnp.int32),
        pltpu.VMEM((NCHUNK, CHUNK), jnp.int32),
        pltpu.VMEM((NCHUNK, CHUNK), jnp.int32),
        pltpu.VMEM((CHUNK, 128), jnp.float32),
        pltpu.VMEM((CHUNK, 128), jnp.float32),
        pltpu.VMEM((CHUNK, 128), jnp.float32),
        pltpu.VMEM((CHUNK, 128), jnp.float32),
        pltpu.VMEM((CHUNK, 128), jnp.float32),
        pltpu.VMEM((CHUNK, 128), jnp.float32),
        pltpu.VMEM((BPW,), jnp.float32),
        pltpu.SemaphoreType.DMA,
    ],
)
def _score(hdiv_h, hcol_h, rdiv_h, rcol_h, tdiv_h, tcol_h,
           ent_h, ent_t, rel, rel_inv,
           out_h, hdiv, hcol, rdiv, rcol, tdiv, tcol,
           hh, ht, th, tt, rv, riv, outv, sem):
    wid = lax.axis_index("s") * NC + lax.axis_index("c")

    pltpu.sync_copy(hdiv_h.at[wid], hdiv)
    pltpu.sync_copy(hcol_h.at[wid], hcol)
    pltpu.sync_copy(rdiv_h.at[wid], rdiv)
    pltpu.sync_copy(rcol_h.at[wid], rcol)
    pltpu.sync_copy(tdiv_h.at[wid], tdiv)
    pltpu.sync_copy(tcol_h.at[wid], tcol)

    lanes = lax.iota(jnp.int32, 16)

    for c in range(NCHUNK):
        copies = [
            pltpu.async_copy(ent_h.at[hdiv.at[c]], hh, sem),
            pltpu.async_copy(ent_h.at[tdiv.at[c]], ht, sem),
            pltpu.async_copy(ent_t.at[hdiv.at[c]], th, sem),
            pltpu.async_copy(ent_t.at[tdiv.at[c]], tt, sem),
            pltpu.async_copy(rel.at[rdiv.at[c]], rv, sem),
            pltpu.async_copy(rel_inv.at[rdiv.at[c]], riv, sem),
        ]
        for cp in copies:
            cp.wait()

        def group(g, carry, c=c):
            rows = g * 16 + lanes
            ch = hcol[c, pl.ds(g * 16, 16)]
            cr = rcol[c, pl.ds(g * 16, 16)]
            ct = tcol[c, pl.ds(g * 16, 16)]
            facc = jnp.zeros((16,), jnp.float32)
            iacc = jnp.zeros((16,), jnp.float32)
            for d in range(EMB_DIM):
                chd = ch + 4 * d
                crd = cr + 4 * d
                ctd = ct + 4 * d
                fh = plsc.load_gather(hh, [rows, chd])
                fr = plsc.load_gather(rv, [rows, crd])
                ft = plsc.load_gather(tt, [rows, ctd])
                facc = facc + fh * fr * ft
                ih = plsc.load_gather(ht, [rows, ctd])
                ir = plsc.load_gather(riv, [rows, crd])
                it = plsc.load_gather(th, [rows, chd])
                iacc = iacc + ih * ir * it
            score = (facc + iacc) * 0.5
            score = jnp.minimum(jnp.maximum(score, -20.0), 20.0)
            outv[pl.ds(c * CHUNK + g * 16, 16)] = score
            return carry

        lax.fori_loop(0, GPC, group, 0)

    pltpu.sync_copy(outv, out_h.at[pl.ds(wid * BPW, BPW)])


def kernel(heads, rels, tails, ent_h_embs, ent_t_embs, rel_embs, rel_inv_embs):
    tail_h = (ent_h_embs[NUM_ENT - 16 * PACK:].reshape(16, PACK, EMB_DIM)
              .transpose(0, 2, 1).reshape(16, 128))
    tail_t = (ent_t_embs[NUM_ENT - 16 * PACK:].reshape(16, PACK, EMB_DIM)
              .transpose(0, 2, 1).reshape(16, 128))
    ent_h_p, ent_t_p = _pack(ent_h_embs.T, ent_t_embs.T, tail_h, tail_t)
    shape3 = (NW, NCHUNK, CHUNK)
    hdiv = (heads // PACK).reshape(shape3)
    hcol = (heads % PACK).reshape(shape3)
    rdiv = (rels // PACK).reshape(shape3)
    rcol = (rels % PACK).reshape(shape3)
    tdiv = (tails // PACK).reshape(shape3)
    tcol = (tails % PACK).reshape(shape3)
    rel_p = (rel_embs.reshape(NUM_REL // PACK, PACK, EMB_DIM)
             .transpose(0, 2, 1).reshape(NUM_REL // PACK, 128))
    rel_inv_p = (rel_inv_embs.reshape(NUM_REL // PACK, PACK, EMB_DIM)
                 .transpose(0, 2, 1).reshape(NUM_REL // PACK, 128))
    return _score(hdiv, hcol, rdiv, rcol, tdiv, tcol,
                  ent_h_p, ent_t_p, rel_p, rel_inv_p)
